# Initial kernel scaffold; baseline (speedup 1.0000x reference)
import functools

import jax
import jax.numpy as jnp
from jax import lax
from jax.experimental import pallas as pl
from jax.experimental.pallas import tpu as pltpu
from jax.experimental.pallas import tpu_sc as plsc

_N, _C, _H, _W = 4, 96, 384, 384
_P = _N * _H * _W
_NW = 32
_PW = _P // _NW
_B = 128
_NCHUNK = _PW // _B


def _sc_body(table, gx_h, gy_h, out_h,
             gxv, gyv, idx0, idx1, idx2, idx3, w0, w1, w2, w3,
             rows0, rows1, rows2, rows3, outbuf, sem):
    cid = lax.axis_index("c")
    sid = lax.axis_index("s")
    wid = sid * 2 + cid
    nbase = (wid // (_H * _W // _PW)) * (_H * _W)
    base = wid * _PW
    lanes = lax.iota(jnp.int32, 16)

    def chunk_body(g, carry):
        cb = base + g * _B
        pltpu.sync_copy(gx_h.at[pl.ds(cb, _B)], gxv)
        pltpu.sync_copy(gy_h.at[pl.ds(cb, _B)], gyv)

        for j in range(_B // 16):
            s = pl.ds(j * 16, 16)
            x = (gxv[s] + 1.0) * (_W * 0.5) - 0.5
            y = (gyv[s] + 1.0) * (_H * 0.5) - 0.5
            xt = x.astype(jnp.int32)
            xtf = xt.astype(jnp.float32)
            x0f = jnp.where(x < xtf, xtf - 1.0, xtf)
            x0 = x0f.astype(jnp.int32)
            yt = y.astype(jnp.int32)
            ytf = yt.astype(jnp.float32)
            y0f = jnp.where(y < ytf, ytf - 1.0, ytf)
            y0 = y0f.astype(jnp.int32)
            x1 = x0 + 1
            y1 = y0 + 1
            wx1 = x - x0f
            wx0 = 1.0 - wx1
            wy1 = y - y0f
            wy0 = 1.0 - wy1
            vx0 = (x0 >= 0) & (x0 <= _W - 1)
            vx1 = (x1 >= 0) & (x1 <= _W - 1)
            vy0 = (y0 >= 0) & (y0 <= _H - 1)
            vy1 = (y1 >= 0) & (y1 <= _H - 1)
            zero = jnp.zeros((16,), jnp.float32)
            xc0 = jnp.clip(x0, 0, _W - 1)
            xc1 = jnp.clip(x1, 0, _W - 1)
            yc0 = jnp.clip(y0, 0, _H - 1)
            yc1 = jnp.clip(y1, 0, _H - 1)
            r0 = nbase + yc0 * _W
            r1 = nbase + yc1 * _W
            idx0[s] = r0 + xc0
            idx1[s] = r0 + xc1
            idx2[s] = r1 + xc0
            idx3[s] = r1 + xc1
            w0[s] = jnp.where(vx0 & vy0, wx0 * wy0, zero)
            w1[s] = jnp.where(vx1 & vy0, wx1 * wy0, zero)
            w2[s] = jnp.where(vx0 & vy1, wx0 * wy1, zero)
            w3[s] = jnp.where(vx1 & vy1, wx1 * wy1, zero)

        cp0 = pltpu.async_copy(table.at[idx0], rows0, sem)
        cp1 = pltpu.async_copy(table.at[idx1], rows1, sem)
        cp2 = pltpu.async_copy(table.at[idx2], rows2, sem)
        cp3 = pltpu.async_copy(table.at[idx3], rows3, sem)
        cp0.wait()
        cp1.wait()
        cp2.wait()
        cp3.wait()

        def acc_body(j, acc_carry):
            off = pl.multiple_of(j * 16, 16)
            s = pl.ds(off, 16)
            w00 = w0[s]
            w01 = w1[s]
            w10 = w2[s]
            w11 = w3[s]
            pix = off + lanes
            for c in range(_C):
                cc = jnp.full((16,), c, jnp.int32)
                v00 = plsc.load_gather(rows0, [pix, cc])
                v01 = plsc.load_gather(rows1, [pix, cc])
                v10 = plsc.load_gather(rows2, [pix, cc])
                v11 = plsc.load_gather(rows3, [pix, cc])
                o = w00 * v00 + w01 * v01 + w10 * v10 + w11 * v11
                plsc.store_scatter(outbuf, [pix, cc], o)
            return acc_carry

        lax.fori_loop(0, _B // 16, acc_body, 0)
        pltpu.sync_copy(outbuf, out_h.at[pl.ds(cb, _B)])
        return carry

    lax.fori_loop(0, _NCHUNK, chunk_body, 0)


_sc_call = functools.partial(
    pl.kernel,
    out_type=jax.ShapeDtypeStruct((_P, _C), jnp.float32),
    mesh=plsc.VectorSubcoreMesh(core_axis_name="c", subcore_axis_name="s"),
    scratch_types=[
        pltpu.VMEM((_B,), jnp.float32),
        pltpu.VMEM((_B,), jnp.float32),
        pltpu.VMEM((_B,), jnp.int32),
        pltpu.VMEM((_B,), jnp.int32),
        pltpu.VMEM((_B,), jnp.int32),
        pltpu.VMEM((_B,), jnp.int32),
        pltpu.VMEM((_B,), jnp.float32),
        pltpu.VMEM((_B,), jnp.float32),
        pltpu.VMEM((_B,), jnp.float32),
        pltpu.VMEM((_B,), jnp.float32),
        pltpu.VMEM((_B, _C), jnp.float32),
        pltpu.VMEM((_B, _C), jnp.float32),
        pltpu.VMEM((_B, _C), jnp.float32),
        pltpu.VMEM((_B, _C), jnp.float32),
        pltpu.VMEM((_B, _C), jnp.float32),
        pltpu.SemaphoreType.DMA,
    ],
)(_sc_body)


@jax.jit
def kernel(input, grid):
    n, c, h, w = input.shape
    assert (n, c, h, w) == (_N, _C, _H, _W)
    table = input.transpose(0, 2, 3, 1).reshape(_P, _C)
    gx = grid[..., 0].reshape(_P)
    gy = grid[..., 1].reshape(_P)
    out = _sc_call(table, gx, gy)
    return out.reshape(_N, _H, _W, _C).transpose(0, 3, 1, 2)


# speedup vs baseline: 1.0358x; 1.0358x over previous
import functools

import jax
import jax.numpy as jnp
from jax import lax
from jax.experimental import pallas as pl
from jax.experimental.pallas import tpu as pltpu
from jax.experimental.pallas import tpu_sc as plsc

_N, _C, _H, _W = 4, 96, 384, 384
_P = _N * _H * _W
_NW = 32
_PW = _P // _NW
_B = 128
_NCHUNK = _PW // _B


def _sc_body(table, gx_h, gy_h, out_h,
             gxv, gyv, idx0, idx1, idx2, idx3, w0, w1, w2, w3,
             rows0, rows1, rows2, rows3, outbuf, sem):
    cid = lax.axis_index("c")
    sid = lax.axis_index("s")
    wid = sid * 2 + cid
    nbase = (wid // (_H * _W // _PW)) * (_H * _W)
    base = wid * _PW
    lanes = lax.iota(jnp.int32, 16)

    def chunk_body(g, carry):
        cb = base + g * _B
        pltpu.sync_copy(gx_h.at[pl.ds(cb, _B)], gxv)
        pltpu.sync_copy(gy_h.at[pl.ds(cb, _B)], gyv)

        for j in range(_B // 16):
            s = pl.ds(j * 16, 16)
            x = (gxv[s] + 1.0) * (_W * 0.5) - 0.5
            y = (gyv[s] + 1.0) * (_H * 0.5) - 0.5
            xt = x.astype(jnp.int32)
            xtf = xt.astype(jnp.float32)
            x0f = jnp.where(x < xtf, xtf - 1.0, xtf)
            x0 = x0f.astype(jnp.int32)
            yt = y.astype(jnp.int32)
            ytf = yt.astype(jnp.float32)
            y0f = jnp.where(y < ytf, ytf - 1.0, ytf)
            y0 = y0f.astype(jnp.int32)
            x1 = x0 + 1
            y1 = y0 + 1
            wx1 = x - x0f
            wx0 = 1.0 - wx1
            wy1 = y - y0f
            wy0 = 1.0 - wy1
            vx0 = (x0 >= 0) & (x0 <= _W - 1)
            vx1 = (x1 >= 0) & (x1 <= _W - 1)
            vy0 = (y0 >= 0) & (y0 <= _H - 1)
            vy1 = (y1 >= 0) & (y1 <= _H - 1)
            zero = jnp.zeros((16,), jnp.float32)
            xc0 = jnp.clip(x0, 0, _W - 1)
            xc1 = jnp.clip(x1, 0, _W - 1)
            yc0 = jnp.clip(y0, 0, _H - 1)
            yc1 = jnp.clip(y1, 0, _H - 1)
            r0 = nbase + yc0 * _W
            r1 = nbase + yc1 * _W
            idx0[s] = r0 + xc0
            idx1[s] = r0 + xc1
            idx2[s] = r1 + xc0
            idx3[s] = r1 + xc1
            w0[s] = jnp.where(vx0 & vy0, wx0 * wy0, zero)
            w1[s] = jnp.where(vx1 & vy0, wx1 * wy0, zero)
            w2[s] = jnp.where(vx0 & vy1, wx0 * wy1, zero)
            w3[s] = jnp.where(vx1 & vy1, wx1 * wy1, zero)

        cp0 = pltpu.async_copy(table.at[idx0], rows0, sem)
        cp1 = pltpu.async_copy(table.at[idx1], rows1, sem)
        cp2 = pltpu.async_copy(table.at[idx2], rows2, sem)
        cp3 = pltpu.async_copy(table.at[idx3], rows3, sem)
        cp0.wait()
        cp1.wait()
        cp2.wait()
        cp3.wait()

        def acc_body(j, acc_carry):
            off = pl.multiple_of(j * 16, 16)
            s16 = pl.ds(off, 16)
            w00v = w0[s16]
            w01v = w1[s16]
            w10v = w2[s16]
            w11v = w3[s16]
            for l in range(16):
                p = off + l
                w00 = jnp.full((16,), w00v[l])
                w01 = jnp.full((16,), w01v[l])
                w10 = jnp.full((16,), w10v[l])
                w11 = jnp.full((16,), w11v[l])
                for cg in range(_C // 16):
                    s = pl.ds(cg * 16, 16)
                    outbuf[p, s] = (w00 * rows0[p, s] + w01 * rows1[p, s]
                                    + w10 * rows2[p, s] + w11 * rows3[p, s])
            return acc_carry

        lax.fori_loop(0, _B // 16, acc_body, 0)
        pltpu.sync_copy(outbuf, out_h.at[pl.ds(cb, _B)])
        return carry

    lax.fori_loop(0, _NCHUNK, chunk_body, 0)


_sc_call = functools.partial(
    pl.kernel,
    out_type=jax.ShapeDtypeStruct((_P, _C), jnp.float32),
    mesh=plsc.VectorSubcoreMesh(core_axis_name="c", subcore_axis_name="s"),
    compiler_params=pltpu.CompilerParams(use_tc_tiling_on_sc=False),
    scratch_types=[
        pltpu.VMEM((_B,), jnp.float32),
        pltpu.VMEM((_B,), jnp.float32),
        pltpu.VMEM((_B,), jnp.int32),
        pltpu.VMEM((_B,), jnp.int32),
        pltpu.VMEM((_B,), jnp.int32),
        pltpu.VMEM((_B,), jnp.int32),
        pltpu.VMEM((_B,), jnp.float32),
        pltpu.VMEM((_B,), jnp.float32),
        pltpu.VMEM((_B,), jnp.float32),
        pltpu.VMEM((_B,), jnp.float32),
        pltpu.VMEM((_B, _C), jnp.float32),
        pltpu.VMEM((_B, _C), jnp.float32),
        pltpu.VMEM((_B, _C), jnp.float32),
        pltpu.VMEM((_B, _C), jnp.float32),
        pltpu.VMEM((_B, _C), jnp.float32),
        pltpu.SemaphoreType.DMA,
    ],
)(_sc_body)


@jax.jit
def kernel(input, grid):
    n, c, h, w = input.shape
    assert (n, c, h, w) == (_N, _C, _H, _W)
    table = input.transpose(0, 2, 3, 1).reshape(_P, _C)
    gx = grid[..., 0].reshape(_P)
    gy = grid[..., 1].reshape(_P)
    out = _sc_call(table, gx, gy)
    return out.reshape(_N, _H, _W, _C).transpose(0, 3, 1, 2)


# speedup vs baseline: 1.2192x; 1.1770x over previous
import functools

import jax
import jax.numpy as jnp
from jax import lax
from jax.experimental import pallas as pl
from jax.experimental.pallas import tpu as pltpu
from jax.experimental.pallas import tpu_sc as plsc

_N, _C, _H, _W = 4, 96, 384, 384
_P = _N * _H * _W
_NW = 32
_PW = _P // _NW
_B = 128
_NCHUNK = _PW // _B


def _sc_body(table, gx_h, gy_h, out_h,
             gxv, gyv,
             idxA, wA, rowsA0, rowsA1, rowsA2, rowsA3,
             idxB, wB, rowsB0, rowsB1, rowsB2, rowsB3,
             outbuf, semA, semB):
    cid = lax.axis_index("c")
    sid = lax.axis_index("s")
    wid = sid * 2 + cid
    nbase = (wid // (_H * _W // _PW)) * (_H * _W)
    base = wid * _PW

    def stage(g, idx4, w4, rows, sem):
        cb = base + g * _B
        pltpu.sync_copy(gx_h.at[pl.ds(cb, _B)], gxv)
        pltpu.sync_copy(gy_h.at[pl.ds(cb, _B)], gyv)
        for j in range(_B // 16):
            s = pl.ds(j * 16, 16)
            x = (gxv[s] + 1.0) * (_W * 0.5) - 0.5
            y = (gyv[s] + 1.0) * (_H * 0.5) - 0.5
            xt = x.astype(jnp.int32)
            xtf = xt.astype(jnp.float32)
            x0f = jnp.where(x < xtf, xtf - 1.0, xtf)
            x0 = x0f.astype(jnp.int32)
            yt = y.astype(jnp.int32)
            ytf = yt.astype(jnp.float32)
            y0f = jnp.where(y < ytf, ytf - 1.0, ytf)
            y0 = y0f.astype(jnp.int32)
            x1 = x0 + 1
            y1 = y0 + 1
            wx1 = x - x0f
            wx0 = 1.0 - wx1
            wy1 = y - y0f
            wy0 = 1.0 - wy1
            vx0 = (x0 >= 0) & (x0 <= _W - 1)
            vx1 = (x1 >= 0) & (x1 <= _W - 1)
            vy0 = (y0 >= 0) & (y0 <= _H - 1)
            vy1 = (y1 >= 0) & (y1 <= _H - 1)
            zero = jnp.zeros((16,), jnp.float32)
            xc0 = jnp.clip(x0, 0, _W - 1)
            xc1 = jnp.clip(x1, 0, _W - 1)
            yc0 = jnp.clip(y0, 0, _H - 1)
            yc1 = jnp.clip(y1, 0, _H - 1)
            r0 = nbase + yc0 * _W
            r1 = nbase + yc1 * _W
            idx4[0, s] = r0 + xc0
            idx4[1, s] = r0 + xc1
            idx4[2, s] = r1 + xc0
            idx4[3, s] = r1 + xc1
            w4[0, s] = jnp.where(vx0 & vy0, wx0 * wy0, zero)
            w4[1, s] = jnp.where(vx1 & vy0, wx1 * wy0, zero)
            w4[2, s] = jnp.where(vx0 & vy1, wx0 * wy1, zero)
            w4[3, s] = jnp.where(vx1 & vy1, wx1 * wy1, zero)
        pltpu.async_copy(table.at[idx4.at[0]], rows[0], sem)
        pltpu.async_copy(table.at[idx4.at[1]], rows[1], sem)
        pltpu.async_copy(table.at[idx4.at[2]], rows[2], sem)
        pltpu.async_copy(table.at[idx4.at[3]], rows[3], sem)

    def finish(g, idx4, w4, rows, sem):
        cb = base + g * _B
        for k in range(4):
            pltpu.make_async_copy(table.at[idx4.at[k]], rows[k], sem).wait()

        def acc_body(j, acc_carry):
            off = pl.multiple_of(j * 16, 16)
            s16 = pl.ds(off, 16)
            w00v = w4[0, s16]
            w01v = w4[1, s16]
            w10v = w4[2, s16]
            w11v = w4[3, s16]
            for l in range(16):
                p = off + l
                w00 = jnp.full((16,), w00v[l])
                w01 = jnp.full((16,), w01v[l])
                w10 = jnp.full((16,), w10v[l])
                w11 = jnp.full((16,), w11v[l])
                for cg in range(_C // 16):
                    s = pl.ds(cg * 16, 16)
                    outbuf[p, s] = (
                        w00 * rows[0][p, s] + w01 * rows[1][p, s]
                        + w10 * rows[2][p, s] + w11 * rows[3][p, s])
            return acc_carry

        lax.fori_loop(0, _B // 16, acc_body, 0)
        pltpu.sync_copy(outbuf, out_h.at[pl.ds(cb, _B)])

    rowsA = (rowsA0, rowsA1, rowsA2, rowsA3)
    rowsB = (rowsB0, rowsB1, rowsB2, rowsB3)

    stage(0, idxA, wA, rowsA, semA)

    def pair_body(i, carry):
        g0 = i * 2
        stage(g0 + 1, idxB, wB, rowsB, semB)
        finish(g0, idxA, wA, rowsA, semA)
        stage(g0 + 2, idxA, wA, rowsA, semA)
        finish(g0 + 1, idxB, wB, rowsB, semB)
        return carry

    lax.fori_loop(0, (_NCHUNK - 2) // 2, pair_body, 0)

    g_last = _NCHUNK - 2
    stage(g_last + 1, idxB, wB, rowsB, semB)
    finish(g_last, idxA, wA, rowsA, semA)
    finish(g_last + 1, idxB, wB, rowsB, semB)


_sc_call = functools.partial(
    pl.kernel,
    out_type=jax.ShapeDtypeStruct((_P, _C), jnp.float32),
    mesh=plsc.VectorSubcoreMesh(core_axis_name="c", subcore_axis_name="s"),
    compiler_params=pltpu.CompilerParams(use_tc_tiling_on_sc=False),
    scratch_types=[
        pltpu.VMEM((_B,), jnp.float32),
        pltpu.VMEM((_B,), jnp.float32),
        pltpu.VMEM((4, _B), jnp.int32),
        pltpu.VMEM((4, _B), jnp.float32),
        pltpu.VMEM((_B, _C), jnp.float32),
        pltpu.VMEM((_B, _C), jnp.float32),
        pltpu.VMEM((_B, _C), jnp.float32),
        pltpu.VMEM((_B, _C), jnp.float32),
        pltpu.VMEM((4, _B), jnp.int32),
        pltpu.VMEM((4, _B), jnp.float32),
        pltpu.VMEM((_B, _C), jnp.float32),
        pltpu.VMEM((_B, _C), jnp.float32),
        pltpu.VMEM((_B, _C), jnp.float32),
        pltpu.VMEM((_B, _C), jnp.float32),
        pltpu.VMEM((_B, _C), jnp.float32),
        pltpu.SemaphoreType.DMA,
        pltpu.SemaphoreType.DMA,
    ],
)(_sc_body)


@jax.jit
def kernel(input, grid):
    n, c, h, w = input.shape
    assert (n, c, h, w) == (_N, _C, _H, _W)
    table = input.transpose(0, 2, 3, 1).reshape(_P, _C)
    gx = grid[..., 0].reshape(_P)
    gy = grid[..., 1].reshape(_P)
    out = _sc_call(table, gx, gy)
    return out.reshape(_N, _H, _W, _C).transpose(0, 3, 1, 2)
